# baseline (speedup 1.0000x reference)
import jax
import jax.numpy as jnp
from jax import lax
from jax.experimental import pallas as pl
from jax.experimental.pallas import tpu as pltpu
from jax.experimental.pallas import tpu_sc as plsc

VOCAB = 1000100
DIM = 64
LANES = 16
NUM_CORES = 2
NUM_SUBCORES = 16
NW = NUM_CORES * NUM_SUBCORES
NTILES = (VOCAB + 127) // 128
NPAD = NTILES * 128
LAST_OFF = VOCAB - 128

_params = pltpu.CompilerParams(
    use_tc_tiling_on_sc=True, needs_layout_passes=False)

NBUF = 3
LAG = 1


def _relayout_body(wt_hbm, tail_hbm, wrm_hbm, *scr):
    tiles = list(scr[0:NBUF])
    trans = list(scr[NBUF:2 * NBUF])
    isems = list(scr[2 * NBUF:3 * NBUF])
    osems = list(scr[3 * NBUF:4 * NBUF])
    wid = lax.axis_index("s") * NUM_CORES + lax.axis_index("c")
    n_units = (NTILES + NW - 1) // NW
    last_t = NTILES - 1

    def off_of(i):
        return pl.multiple_of((i * NW + wid) * 128, 128)

    def start_in(i, b):
        t = i * NW + wid

        @pl.when(t < last_t)
        def _full():
            pltpu.async_copy(wt_hbm.at[:, pl.ds(off_of(i), 128)],
                             tiles[b], isems[b])

        @pl.when(t == last_t)
        def _tail():
            pltpu.async_copy(tail_hbm, trans[b], isems[b])

    def wait_in(j, b):
        t = j * NW + wid

        @pl.when(t < last_t)
        def _full():
            pltpu.make_async_copy(wt_hbm.at[:, pl.ds(0, 128)],
                                  tiles[b], isems[b]).wait()

        @pl.when(t == last_t)
        def _tail():
            pltpu.make_async_copy(tail_hbm, trans[b], isems[b]).wait()

    def start_out(i, b):
        pltpu.async_copy(trans[b], wrm_hbm.at[pl.ds(off_of(i), 128)],
                         osems[b])

    def wait_out(b):
        pltpu.make_async_copy(trans[b], wrm_hbm.at[pl.ds(0, 128)],
                              osems[b]).wait()

    iv = lax.iota(jnp.int32, LANES)

    def transpose(b):
        tile_v, trans_v = tiles[b], trans[b]

        def row(d, _):
            dv = jnp.full((LANES,), 0, jnp.int32) + d
            for k in range(8):
                v = tile_v[d, pl.ds(k * LANES, LANES)]
                plsc.store_scatter(trans_v, [iv + k * LANES, dv], v)
            return 0

        lax.fori_loop(0, DIM, row, 0, unroll=False)

    def active(i):
        return jnp.logical_and(i >= 0, i * NW + wid < NTILES)

    n_groups = (n_units + LAG) // NBUF + 1

    def group(g, _):
        for b in range(NBUF):
            i = g * NBUF + b

            @pl.when(active(i))
            def _in():
                @pl.when(i >= NBUF)
                def _drain():
                    wait_out(b)

                start_in(i, b)

            j = i - LAG
            bj = (b + NBUF - LAG) % NBUF

            @pl.when(active(j))
            def _compute():
                wait_in(j, bj)

                @pl.when(j * NW + wid < last_t)
                def _tr():
                    transpose(bj)

                start_out(j, bj)

        return 0

    lax.fori_loop(0, n_groups, group, 0, unroll=False)

    for b in range(NBUF):
        @pl.when(active(b))
        def _final():
            wait_out(b)


def _gather_body(ids_hbm, w_hbm, out_hbm, *scr):
    idx_v = scr[0]
    rows = list(scr[1:1 + NBUF])
    trans = list(scr[1 + NBUF:1 + 2 * NBUF])
    gsems = list(scr[1 + 2 * NBUF:1 + 3 * NBUF])
    osems = list(scr[1 + 3 * NBUF:1 + 4 * NBUF])
    wid = lax.axis_index("s") * NUM_CORES + lax.axis_index("c")
    hist = idx_v.shape[0]

    pltpu.sync_copy(
        ids_hbm.at[:, pl.ds(pl.multiple_of(wid * 128, 128), 128)], idx_v)

    def clamp_row(h, _):
        for j in range(128 // LANES):
            sl = pl.ds(j * LANES, LANES)
            idx_v[h, sl] = jnp.minimum(idx_v[h, sl], VOCAB - 1)
        return 0

    lax.fori_loop(0, hist, clamp_row, 0, unroll=False)

    def start_gather(h, b):
        pltpu.async_copy(w_hbm.at[idx_v.at[h]], rows[b], gsems[b])

    def wait_gather(b):
        pltpu.make_async_copy(w_hbm.at[pl.ds(0, 128)], rows[b],
                              gsems[b]).wait()

    def start_out(h, b):
        for dt in range(DIM // 8):
            pltpu.async_copy(trans[b].at[pl.ds(dt * 8, 8)],
                             out_hbm.at[h, dt, wid], osems[b])

    def wait_out(b):
        for dt in range(DIM // 8):
            pltpu.make_async_copy(trans[b].at[pl.ds(dt * 8, 8)],
                                  out_hbm.at[0, 0, 0], osems[b]).wait()

    iv = lax.iota(jnp.int32, LANES)

    def transpose(b):
        rows_v, trans_v = rows[b], trans[b]

        def col(c, _):
            cv = jnp.full((LANES,), 0, jnp.int32) + c
            for k in range(DIM // LANES):
                v = rows_v[c, pl.ds(k * LANES, LANES)]
                plsc.store_scatter(trans_v, [iv + k * LANES, cv], v)
            return 0

        lax.fori_loop(0, 128, col, 0, unroll=False)

    n_groups = (hist + LAG) // NBUF + 1

    def group(g, _):
        for b in range(NBUF):
            h = g * NBUF + b

            @pl.when(h < hist)
            def _in():
                @pl.when(h >= NBUF)
                def _drain():
                    wait_out(b)

                start_gather(h, b)

            j = h - LAG
            bj = (b + NBUF - LAG) % NBUF

            @pl.when(jnp.logical_and(j >= 0, j < hist))
            def _compute():
                wait_gather(bj)
                transpose(bj)
                start_out(j, bj)

        return 0

    lax.fori_loop(0, n_groups, group, 0, unroll=False)

    for b in range(NBUF):
        wait_out(b)


def kernel(input_ids, weight):
    batch, hist = input_ids.shape
    assert batch == NW * 128
    idsT = input_ids.astype(jnp.int32).T
    wt = weight.T
    tail = jnp.pad(weight[(NTILES - 1) * 128:],
                   ((0, NPAD - VOCAB), (0, 128 - DIM)))

    mesh = plsc.VectorSubcoreMesh(
        core_axis_name="c", subcore_axis_name="s",
        num_cores=NUM_CORES, num_subcores=NUM_SUBCORES)

    wrm = pl.kernel(
        _relayout_body,
        out_type=jax.ShapeDtypeStruct((NPAD, 128), jnp.float32),
        mesh=mesh,
        scratch_types=(
            [pltpu.VMEM((DIM, 128), jnp.float32) for _ in range(NBUF)]
            + [pltpu.VMEM((128, 128), jnp.float32) for _ in range(NBUF)]
            + [pltpu.SemaphoreType.DMA for _ in range(2 * NBUF)]
        ),
        compiler_params=_params,
    )(wt, tail)

    out5 = pl.kernel(
        _gather_body,
        out_type=jax.ShapeDtypeStruct(
            (hist, DIM // 8, batch // 128, 8, 128), jnp.float32),
        mesh=mesh,
        scratch_types=(
            [pltpu.VMEM((hist, 128), jnp.int32)]
            + [pltpu.VMEM((128, 128), jnp.float32) for _ in range(NBUF)]
            + [pltpu.VMEM((DIM, 128), jnp.float32) for _ in range(NBUF)]
            + [pltpu.SemaphoreType.DMA for _ in range(2 * NBUF)]
        ),
        compiler_params=_params,
    )(idsT, wrm)
    return out5.transpose(2, 4, 0, 1, 3).reshape(batch, hist, DIM)


# speedup vs baseline: 1.9699x; 1.9699x over previous
import jax
import jax.numpy as jnp
from jax import lax
from jax.experimental import pallas as pl
from jax.experimental.pallas import tpu as pltpu
from jax.experimental.pallas import tpu_sc as plsc

VOCAB = 1000100
DIM = 64
LANES = 16
NUM_CORES = 2
NUM_SUBCORES = 16
NW = NUM_CORES * NUM_SUBCORES
NTILES = (VOCAB + 127) // 128
NPAD = NTILES * 128
LAST_OFF = VOCAB - 128

_params = pltpu.CompilerParams(
    use_tc_tiling_on_sc=True, needs_layout_passes=False)

NBUF = 3
LAG = 1


def _relayout_body(wt_hbm, tail_hbm, wrm_hbm, *scr):
    tiles = list(scr[0:NBUF])
    trans = list(scr[NBUF:2 * NBUF])
    isems = list(scr[2 * NBUF:3 * NBUF])
    osems = list(scr[3 * NBUF:4 * NBUF])
    wid = lax.axis_index("s") * NUM_CORES + lax.axis_index("c")
    n_units = (NTILES + NW - 1) // NW
    last_t = NTILES - 1

    def off_of(i):
        return pl.multiple_of((i * NW + wid) * 128, 128)

    def start_in(i, b):
        t = i * NW + wid

        @pl.when(t < last_t)
        def _full():
            pltpu.async_copy(wt_hbm.at[:, pl.ds(off_of(i), 128)],
                             tiles[b], isems[b])

        @pl.when(t == last_t)
        def _tail():
            pltpu.async_copy(tail_hbm, trans[b], isems[b])

    def wait_in(j, b):
        t = j * NW + wid

        @pl.when(t < last_t)
        def _full():
            pltpu.make_async_copy(wt_hbm.at[:, pl.ds(0, 128)],
                                  tiles[b], isems[b]).wait()

        @pl.when(t == last_t)
        def _tail():
            pltpu.make_async_copy(tail_hbm, trans[b], isems[b]).wait()

    def start_out(i, b):
        pltpu.async_copy(trans[b], wrm_hbm.at[pl.ds(off_of(i), 128)],
                         osems[b])

    def wait_out(b):
        pltpu.make_async_copy(trans[b], wrm_hbm.at[pl.ds(0, 128)],
                              osems[b]).wait()

    iv = lax.iota(jnp.int32, LANES)

    def transpose(b):
        tile_v, trans_v = tiles[b], trans[b]

        def cblock(cb, _):
            for db in range(DIM // LANES):
                dd = db * LANES + iv
                for s in range(LANES):
                    cc = cb * LANES + ((iv + s) & (LANES - 1))
                    v = plsc.load_gather(tile_v, [dd, cc])
                    plsc.store_scatter(trans_v, [cc, dd], v)
            return 0

        lax.fori_loop(0, 128 // LANES, cblock, 0, unroll=False)

    def active(i):
        return jnp.logical_and(i >= 0, i * NW + wid < NTILES)

    n_groups = (n_units + LAG) // NBUF + 1

    def group(g, _):
        for b in range(NBUF):
            i = g * NBUF + b

            @pl.when(active(i))
            def _in():
                @pl.when(i >= NBUF)
                def _drain():
                    wait_out(b)

                start_in(i, b)

            j = i - LAG
            bj = (b + NBUF - LAG) % NBUF

            @pl.when(active(j))
            def _compute():
                wait_in(j, bj)

                @pl.when(j * NW + wid < last_t)
                def _tr():
                    transpose(bj)

                start_out(j, bj)

        return 0

    lax.fori_loop(0, n_groups, group, 0, unroll=False)

    for b in range(NBUF):
        @pl.when(active(b))
        def _final():
            wait_out(b)


def _gather_body(ids_hbm, w_hbm, out_hbm, *scr):
    idx_v = scr[0]
    rows = list(scr[1:1 + NBUF])
    trans = list(scr[1 + NBUF:1 + 2 * NBUF])
    gsems = list(scr[1 + 2 * NBUF:1 + 3 * NBUF])
    osems = list(scr[1 + 3 * NBUF:1 + 4 * NBUF])
    wid = lax.axis_index("s") * NUM_CORES + lax.axis_index("c")
    hist = idx_v.shape[0]

    pltpu.sync_copy(
        ids_hbm.at[:, pl.ds(pl.multiple_of(wid * 128, 128), 128)], idx_v)

    def clamp_row(h, _):
        for j in range(128 // LANES):
            sl = pl.ds(j * LANES, LANES)
            idx_v[h, sl] = jnp.minimum(idx_v[h, sl], VOCAB - 1)
        return 0

    lax.fori_loop(0, hist, clamp_row, 0, unroll=False)

    def start_gather(h, b):
        pltpu.async_copy(w_hbm.at[idx_v.at[h]], rows[b], gsems[b])

    def wait_gather(b):
        pltpu.make_async_copy(w_hbm.at[pl.ds(0, 128)], rows[b],
                              gsems[b]).wait()

    def start_out(h, b):
        for dt in range(DIM // 8):
            pltpu.async_copy(trans[b].at[pl.ds(dt * 8, 8)],
                             out_hbm.at[h, dt, wid], osems[b])

    def wait_out(b):
        for dt in range(DIM // 8):
            pltpu.make_async_copy(trans[b].at[pl.ds(dt * 8, 8)],
                                  out_hbm.at[0, 0, 0], osems[b]).wait()

    iv = lax.iota(jnp.int32, LANES)

    def transpose(b):
        rows_v, trans_v = rows[b], trans[b]

        def cblock(cb, _):
            for db in range(DIM // LANES):
                dd = db * LANES + iv
                for s in range(LANES):
                    cc = cb * LANES + ((iv + s) & (LANES - 1))
                    v = plsc.load_gather(rows_v, [cc, dd])
                    plsc.store_scatter(trans_v, [dd, cc], v)
            return 0

        lax.fori_loop(0, 128 // LANES, cblock, 0, unroll=False)

    n_groups = (hist + LAG) // NBUF + 1

    def group(g, _):
        for b in range(NBUF):
            h = g * NBUF + b

            @pl.when(h < hist)
            def _in():
                @pl.when(h >= NBUF)
                def _drain():
                    wait_out(b)

                start_gather(h, b)

            j = h - LAG
            bj = (b + NBUF - LAG) % NBUF

            @pl.when(jnp.logical_and(j >= 0, j < hist))
            def _compute():
                wait_gather(bj)
                transpose(bj)
                start_out(j, bj)

        return 0

    lax.fori_loop(0, n_groups, group, 0, unroll=False)

    for b in range(NBUF):
        wait_out(b)


def kernel(input_ids, weight):
    batch, hist = input_ids.shape
    assert batch == NW * 128
    idsT = input_ids.astype(jnp.int32).T
    wt = weight.T
    tail = jnp.pad(weight[(NTILES - 1) * 128:],
                   ((0, NPAD - VOCAB), (0, 128 - DIM)))

    mesh = plsc.VectorSubcoreMesh(
        core_axis_name="c", subcore_axis_name="s",
        num_cores=NUM_CORES, num_subcores=NUM_SUBCORES)

    wrm = pl.kernel(
        _relayout_body,
        out_type=jax.ShapeDtypeStruct((NPAD, 128), jnp.float32),
        mesh=mesh,
        scratch_types=(
            [pltpu.VMEM((DIM, 128), jnp.float32) for _ in range(NBUF)]
            + [pltpu.VMEM((128, 128), jnp.float32) for _ in range(NBUF)]
            + [pltpu.SemaphoreType.DMA for _ in range(2 * NBUF)]
        ),
        compiler_params=_params,
    )(wt, tail)

    out5 = pl.kernel(
        _gather_body,
        out_type=jax.ShapeDtypeStruct(
            (hist, DIM // 8, batch // 128, 8, 128), jnp.float32),
        mesh=mesh,
        scratch_types=(
            [pltpu.VMEM((hist, 128), jnp.int32)]
            + [pltpu.VMEM((128, 128), jnp.float32) for _ in range(NBUF)]
            + [pltpu.VMEM((DIM, 128), jnp.float32) for _ in range(NBUF)]
            + [pltpu.SemaphoreType.DMA for _ in range(2 * NBUF)]
        ),
        compiler_params=_params,
    )(idsT, wrm)
    return out5.transpose(2, 4, 0, 1, 3).reshape(batch, hist, DIM)
